# Initial kernel scaffold; baseline (speedup 1.0000x reference)
import jax
import jax.numpy as jnp
from jax.experimental import pallas as pl


def kernel(h, edge_index, ppr_weight, W1, b1, W2, b2):
    raise NotImplementedError("write your pallas kernel here")


# speedup vs baseline: 4.4526x; 4.4526x over previous
import functools

import jax
import jax.numpy as jnp
from jax import lax
from jax.experimental import pallas as pl
from jax.experimental.pallas import tpu as pltpu
from jax.experimental.pallas import tpu_sc as plsc

_N = 10000
_E = 160000
_D = 256
_H = 144
_K = 80
_C = 125
_T = 16
_RPT = _N // _T
_NP = 10240
_BK = 512


def _sc_agg(h_aug, pk3, ppr3):
    mesh = plsc.VectorSubcoreMesh(core_axis_name="c", subcore_axis_name="s")

    @functools.partial(
        pl.kernel,
        mesh=mesh,
        compiler_params=pltpu.CompilerParams(use_tc_tiling_on_sc=False),
        out_type=jax.ShapeDtypeStruct((2, _N, _H), jnp.float32),
        scratch_types=[
            pltpu.VMEM((4, _K), jnp.int32),
            pltpu.VMEM((4, _K), jnp.float32),
            pltpu.VMEM((2, _K), jnp.int32),
            pltpu.VMEM((2, _K), jnp.int32),
            pltpu.VMEM((2, _K, _H), jnp.float32),
            pltpu.VMEM_SHARED((_N, _H), jnp.float32),
            pltpu.SemaphoreType.DMA,
            pltpu.SemaphoreType.DMA,
            pltpu.SemaphoreType.DMA,
            pltpu.SemaphoreType.DMA,
            pltpu.SemaphoreType.DMA,
            pltpu.SemaphoreType.DMA,
            pltpu.SemaphoreType.DMA,
            pltpu.SemaphoreType.DMA,
            pltpu.SemaphoreType.DMA,
            pltpu.SemaphoreType.DMA,
            pltpu.SemaphoreType.DMA,
            pltpu.SemaphoreType.DMA,
        ],
    )
    def k(h_hbm, pk_hbm, ppr_hbm, out_hbm,
          pk_v, ppr_v, idx_v, dst_v, rows_v, acc_sh,
          ek0, ek1, ek2, ek3, ep0, ep1, ep2, ep3, gs0, gs1, ss0, ss1):
        c = lax.axis_index("c")
        s = lax.axis_index("s")
        cofs = c * _N
        eks = (ek0, ek1, ek2, ek3)
        eps = (ep0, ep1, ep2, ep3)
        gss = (gs0, gs1)
        sss = (ss0, ss1)

        def start_edge(t, e4):
            pltpu.async_copy(pk_hbm.at[s, t], pk_v.at[e4], eks[e4])
            pltpu.async_copy(ppr_hbm.at[s, t], ppr_v.at[e4], eps[e4])

        def wait_edge(e4):
            pltpu.make_async_copy(pk_hbm.at[s, 0], pk_v.at[e4], eks[e4]).wait()
            pltpu.make_async_copy(ppr_hbm.at[s, 0], ppr_v.at[e4], eps[e4]).wait()

        def unpack(e4, b):
            for u in range(_K // 16):
                sl = pl.ds(u * 16, 16)
                pk = pk_v[e4, sl]
                idx_v[b, sl] = (pk & 0xFFFF) + cofs
                dst_v[b, sl] = pk >> 16

        def start_gather(b):
            pltpu.async_copy(h_hbm.at[idx_v.at[b]], rows_v.at[b], gss[b])

        def wait_gather(b):
            pltpu.make_async_copy(
                h_hbm.at[idx_v.at[b]], rows_v.at[b], gss[b]).wait()

        def start_scatter(b):
            pltpu.async_copy(rows_v.at[b], acc_sh.at[dst_v.at[b]], sss[b],
                             add=True)

        def wait_scatter(b):
            pltpu.make_async_copy(
                rows_v.at[b], acc_sh.at[dst_v.at[b]], sss[b]).wait()

        def scale(e4, b):
            def sgroup(g, gcarry):
                pw = ppr_v[e4, pl.ds(g * 16, 16)]
                rbase = g * 16
                for rr in range(16):
                    pv = jnp.full((16,), pw[rr], jnp.float32)
                    for u in range(_H // 16):
                        sl = pl.ds(u * 16, 16)
                        rows_v[b, rbase + rr, sl] = (
                            rows_v[b, rbase + rr, sl] * pv)
                return gcarry

            lax.fori_loop(0, _K // 16, sgroup, 0)

        zero16f = jnp.zeros((16,), jnp.float32)
        zero16i = jnp.zeros((16,), jnp.int32)

        def zrow(r, carry):
            for b in range(2):
                for u in range(_H // 16):
                    rows_v[b, r, pl.ds(u * 16, 16)] = zero16f
            return carry

        lax.fori_loop(0, _K, zrow, 0)
        for u in range(_K // 16):
            dst_v[0, pl.ds(u * 16, 16)] = zero16i
            dst_v[1, pl.ds(u * 16, 16)] = zero16i

        base = s * _RPT
        for blk in range(_RPT // _K):
            pltpu.sync_copy(rows_v.at[0], acc_sh.at[pl.ds(base + blk * _K, _K)])
        pltpu.sync_copy(rows_v.at[0, pl.ds(0, _RPT % _K)],
                        acc_sh.at[pl.ds(base + (_RPT // _K) * _K, _RPT % _K)])

        for t0 in range(4):
            start_edge(t0, t0)

        plsc.subcore_barrier()

        start_scatter(1)

        wait_edge(0)
        unpack(0, 0)
        start_gather(0)

        def step(t, e4, e4n, b, bn, prefetch):
            wait_gather(b)
            if prefetch:
                wait_edge(e4n)
                wait_scatter(bn)
                unpack(e4n, bn)
                start_gather(bn)
            scale(e4, b)
            if prefetch:

                @pl.when(t + 4 < _C)
                def _():
                    start_edge(t + 4, e4)

            start_scatter(b)

        def quad(i, carry):
            t = i * 4
            step(t, 0, 1, 0, 1, True)
            step(t + 1, 1, 2, 1, 0, True)
            step(t + 2, 2, 3, 0, 1, True)
            step(t + 3, 3, 0, 1, 0, True)
            return carry

        lax.fori_loop(0, _C // 4 - 1, quad, 0)

        t = _C - 5
        step(t, 0, 1, 0, 1, True)
        step(t + 1, 1, 2, 1, 0, True)
        step(t + 2, 2, 3, 0, 1, True)
        step(t + 3, 3, 0, 1, 0, True)
        step(t + 4, 0, 1, 0, 1, False)

        wait_scatter(1)
        wait_scatter(0)

        plsc.subcore_barrier()

        _ZR = 125
        for blk in range(_RPT // _ZR):
            ofs = base + blk * _ZR
            pltpu.sync_copy(acc_sh.at[pl.ds(ofs, _ZR)],
                            out_hbm.at[c, pl.ds(ofs, _ZR)])

    return k(h_aug, pk3, ppr3)


def _tc_body(hh_ref, alo_ref, ahi_ref, w_ref, a_ref, blo_ref, bhi_ref,
             b_ref, o_ref):
    w = w_ref[...]
    inv = 1.0 / jnp.where(w == 0.0, 1.0, w)
    alo = alo_ref[...] * inv
    ahi = ahi_ref[...] * inv
    acc = (jnp.dot(hh_ref[...], a_ref[...], preferred_element_type=jnp.float32)
           + jnp.dot(alo, blo_ref[...], preferred_element_type=jnp.float32)
           + jnp.dot(ahi, bhi_ref[...], preferred_element_type=jnp.float32)
           + b_ref[...])
    y = jnp.where(acc > 0.0, acc, 0.01 * acc)
    ss = jnp.sum(y * y, axis=1, keepdims=True)
    nrm = jnp.sqrt(ss)
    nrm = jnp.where(nrm == 0.0, 1.0, nrm)
    o_ref[...] = y / nrm


def _tc_update(hh, alo, ahi, w, a, blo, bhi, b):
    grid = (_NP // _BK,)
    return pl.pallas_call(
        _tc_body,
        grid=grid,
        in_specs=[
            pl.BlockSpec((_BK, _D), lambda i: (i, 0)),
            pl.BlockSpec((_BK, 128), lambda i: (i, 0)),
            pl.BlockSpec((_BK, 128), lambda i: (i, 0)),
            pl.BlockSpec((_BK, 1), lambda i: (i, 0)),
            pl.BlockSpec((_D, _D), lambda i: (0, 0)),
            pl.BlockSpec((128, _D), lambda i: (0, 0)),
            pl.BlockSpec((128, _D), lambda i: (0, 0)),
            pl.BlockSpec((1, _D), lambda i: (0, 0)),
        ],
        out_specs=pl.BlockSpec((_BK, _D), lambda i: (i, 0)),
        out_shape=jax.ShapeDtypeStruct((_NP, _D), jnp.float32),
    )(hh, alo, ahi, w, a, blo, bhi, b)


def _to_aug(hh):
    ones = jnp.ones((_N, 1), jnp.float32)
    z = jnp.zeros((_N, _H - 129), jnp.float32)
    lo = jnp.concatenate([hh[:, :128], ones, z], axis=1)
    hi = jnp.concatenate([hh[:, 128:], ones, z], axis=1)
    return jnp.concatenate([lo, hi], axis=0)


def _pad_rows(x):
    return jnp.pad(x, ((0, _NP - _N), (0, 0)))


def kernel(h, edge_index, ppr_weight, W1, b1, W2, b2):
    pk3 = (edge_index[0] | (edge_index[1] << 16)).reshape(_T, _C, _K)
    ppr3 = ppr_weight.reshape(_T, _C, _K)

    hh = h
    for (W, b) in ((W1, b1), (W2, b2)):
        agg = _sc_agg(_to_aug(hh), pk3, ppr3)
        alo = agg[0, :, :128]
        ahi = agg[1, :, :128]
        w = agg[0, :, 128:129]
        wt = W.T
        hh = _tc_update(
            _pad_rows(hh), _pad_rows(alo), _pad_rows(ahi), _pad_rows(w),
            wt[:_D], wt[_D:_D + 128], wt[_D + 128:], b.reshape(1, _D),
        )[:_N]
    return hh


# speedup vs baseline: 5.0718x; 1.1391x over previous
import functools

import jax
import jax.numpy as jnp
from jax import lax
from jax.experimental import pallas as pl
from jax.experimental.pallas import tpu as pltpu
from jax.experimental.pallas import tpu_sc as plsc

_N = 10000
_E = 160000
_D = 256
_H = 144
_K = 80
_C = 125
_T = 16
_RPT = _N // _T
_BK = 400


def _sc_agg(h_aug, pk3, ppr3):
    mesh = plsc.VectorSubcoreMesh(core_axis_name="c", subcore_axis_name="s")

    @functools.partial(
        pl.kernel,
        mesh=mesh,
        compiler_params=pltpu.CompilerParams(use_tc_tiling_on_sc=False),
        out_type=jax.ShapeDtypeStruct((2, _N, _H), jnp.float32),
        scratch_types=[
            pltpu.VMEM((4, _K), jnp.int32),
            pltpu.VMEM((4, _K), jnp.float32),
            pltpu.VMEM((2, _K), jnp.int32),
            pltpu.VMEM((2, _K), jnp.int32),
            pltpu.VMEM((2, _K, _H), jnp.float32),
            pltpu.VMEM_SHARED((_N, _H), jnp.float32),
            pltpu.SemaphoreType.DMA,
            pltpu.SemaphoreType.DMA,
            pltpu.SemaphoreType.DMA,
            pltpu.SemaphoreType.DMA,
            pltpu.SemaphoreType.DMA,
            pltpu.SemaphoreType.DMA,
            pltpu.SemaphoreType.DMA,
            pltpu.SemaphoreType.DMA,
            pltpu.SemaphoreType.DMA,
            pltpu.SemaphoreType.DMA,
            pltpu.SemaphoreType.DMA,
            pltpu.SemaphoreType.DMA,
        ],
    )
    def k(h_hbm, pk_hbm, ppr_hbm, out_hbm,
          pk_v, ppr_v, idx_v, dst_v, rows_v, acc_sh,
          ek0, ek1, ek2, ek3, ep0, ep1, ep2, ep3, gs0, gs1, ss0, ss1):
        c = lax.axis_index("c")
        s = lax.axis_index("s")
        cofs = c * _N
        eks = (ek0, ek1, ek2, ek3)
        eps = (ep0, ep1, ep2, ep3)
        gss = (gs0, gs1)
        sss = (ss0, ss1)

        def start_edge(t, e4):
            pltpu.async_copy(pk_hbm.at[s, t], pk_v.at[e4], eks[e4])
            pltpu.async_copy(ppr_hbm.at[s, t], ppr_v.at[e4], eps[e4])

        def wait_edge(e4):
            pltpu.make_async_copy(pk_hbm.at[s, 0], pk_v.at[e4], eks[e4]).wait()
            pltpu.make_async_copy(ppr_hbm.at[s, 0], ppr_v.at[e4], eps[e4]).wait()

        def unpack(e4, b):
            for u in range(_K // 16):
                sl = pl.ds(u * 16, 16)
                pk = pk_v[e4, sl]
                idx_v[b, sl] = (pk & 0xFFFF) + cofs
                dst_v[b, sl] = pk >> 16

        def start_gather(b):
            pltpu.async_copy(h_hbm.at[idx_v.at[b]], rows_v.at[b], gss[b])

        def wait_gather(b):
            pltpu.make_async_copy(
                h_hbm.at[idx_v.at[b]], rows_v.at[b], gss[b]).wait()

        def start_scatter(b):
            pltpu.async_copy(rows_v.at[b], acc_sh.at[dst_v.at[b]], sss[b],
                             add=True)

        def wait_scatter(b):
            pltpu.make_async_copy(
                rows_v.at[b], acc_sh.at[dst_v.at[b]], sss[b]).wait()

        def scale(e4, b):
            def sgroup(g, gcarry):
                pw = ppr_v[e4, pl.ds(g * 16, 16)]
                rbase = g * 16
                for rr in range(16):
                    pv = jnp.full((16,), pw[rr], jnp.float32)
                    for u in range(_H // 16):
                        sl = pl.ds(u * 16, 16)
                        rows_v[b, rbase + rr, sl] = (
                            rows_v[b, rbase + rr, sl] * pv)
                return gcarry

            lax.fori_loop(0, _K // 16, sgroup, 0)

        zero16f = jnp.zeros((16,), jnp.float32)
        zero16i = jnp.zeros((16,), jnp.int32)

        def zrow(r, carry):
            for b in range(2):
                for u in range(_H // 16):
                    rows_v[b, r, pl.ds(u * 16, 16)] = zero16f
            return carry

        lax.fori_loop(0, _K, zrow, 0)
        for u in range(_K // 16):
            dst_v[0, pl.ds(u * 16, 16)] = zero16i
            dst_v[1, pl.ds(u * 16, 16)] = zero16i

        base = s * _RPT
        for blk in range(_RPT // _K):
            pltpu.sync_copy(rows_v.at[0], acc_sh.at[pl.ds(base + blk * _K, _K)])
        pltpu.sync_copy(rows_v.at[0, pl.ds(0, _RPT % _K)],
                        acc_sh.at[pl.ds(base + (_RPT // _K) * _K, _RPT % _K)])

        for t0 in range(4):
            start_edge(t0, t0)

        plsc.subcore_barrier()

        start_scatter(1)

        wait_edge(0)
        unpack(0, 0)
        start_gather(0)

        def step(t, e4, e4n, b, bn, prefetch):
            wait_gather(b)
            if prefetch:
                wait_edge(e4n)
                wait_scatter(bn)
                unpack(e4n, bn)
                start_gather(bn)
            scale(e4, b)
            if prefetch:

                @pl.when(t + 4 < _C)
                def _():
                    start_edge(t + 4, e4)

            start_scatter(b)

        def quad(i, carry):
            t = i * 4
            step(t, 0, 1, 0, 1, True)
            step(t + 1, 1, 2, 1, 0, True)
            step(t + 2, 2, 3, 0, 1, True)
            step(t + 3, 3, 0, 1, 0, True)
            return carry

        lax.fori_loop(0, _C // 4 - 1, quad, 0)

        t = _C - 5
        step(t, 0, 1, 0, 1, True)
        step(t + 1, 1, 2, 1, 0, True)
        step(t + 2, 2, 3, 0, 1, True)
        step(t + 3, 3, 0, 1, 0, True)
        step(t + 4, 0, 1, 0, 1, False)

        wait_scatter(1)
        wait_scatter(0)

        plsc.subcore_barrier()

        _ZR = 125
        for blk in range(_RPT // _ZR):
            ofs = base + blk * _ZR
            pltpu.sync_copy(acc_sh.at[pl.ds(ofs, _ZR)],
                            out_hbm.at[c, pl.ds(ofs, _ZR)])

    return k(h_aug, pk3, ppr3)


def _aug_cols(bk):
    col = lax.broadcasted_iota(jnp.int32, (bk, _H - 128), 1)
    return jnp.where(col == 0, 1.0, 0.0).astype(jnp.float32)


def _tc_body(hh_ref, agg_ref, ws_ref, wl_ref, wh_ref, b_ref, o_ref,
             aux_ref=None):
    dn = (((1,), (1,)), ((), ()))
    w = agg_ref[0, :, 128:129]
    inv = 1.0 / jnp.where(w == 0.0, 1.0, w)
    alo = agg_ref[0, :, 0:128] * inv
    ahi = agg_ref[1, :, 0:128] * inv
    acc = (lax.dot_general(hh_ref[...], ws_ref[...], dn,
                           preferred_element_type=jnp.float32)
           + lax.dot_general(alo, wl_ref[...], dn,
                             preferred_element_type=jnp.float32)
           + lax.dot_general(ahi, wh_ref[...], dn,
                             preferred_element_type=jnp.float32)
           + b_ref[...])
    y = jnp.where(acc > 0.0, acc, 0.01 * acc)
    ss = jnp.sum(y * y, axis=1, keepdims=True)
    nrm = jnp.sqrt(ss)
    nrm = jnp.where(nrm == 0.0, 1.0, nrm)
    res = y / nrm
    o_ref[...] = res
    if aux_ref is not None:
        tail = _aug_cols(res.shape[0])
        aux_ref[0, :, 0:128] = res[:, 0:128]
        aux_ref[1, :, 0:128] = res[:, 128:256]
        aux_ref[0, :, 128:_H] = tail
        aux_ref[1, :, 128:_H] = tail


def _tc_update(hh, agg, W, b, make_aux):
    grid = (_N // _BK,)
    in_specs = [
        pl.BlockSpec((_BK, _D), lambda i: (i, 0)),
        pl.BlockSpec((2, _BK, _H), lambda i: (0, i, 0)),
        pl.BlockSpec((_D, _D), lambda i: (0, 0)),
        pl.BlockSpec((_D, 128), lambda i: (0, 2)),
        pl.BlockSpec((_D, 128), lambda i: (0, 3)),
        pl.BlockSpec((1, _D), lambda i: (0, 0)),
    ]
    out_shape = [jax.ShapeDtypeStruct((_N, _D), jnp.float32)]
    out_specs = [pl.BlockSpec((_BK, _D), lambda i: (i, 0))]
    if make_aux:
        out_shape.append(jax.ShapeDtypeStruct((2, _N, _H), jnp.float32))
        out_specs.append(pl.BlockSpec((2, _BK, _H), lambda i: (0, i, 0)))
    return pl.pallas_call(
        _tc_body,
        grid=grid,
        in_specs=in_specs,
        out_specs=out_specs,
        out_shape=out_shape,
    )(hh, agg, W, W, W, b.reshape(1, _D))


def _prep_body(h_ref, o_ref):
    tail = _aug_cols(h_ref.shape[0])
    o_ref[0, :, 0:128] = h_ref[:, 0:128]
    o_ref[1, :, 0:128] = h_ref[:, 128:256]
    o_ref[0, :, 128:_H] = tail
    o_ref[1, :, 128:_H] = tail


def _prep(h):
    return pl.pallas_call(
        _prep_body,
        grid=(_N // _BK,),
        in_specs=[pl.BlockSpec((_BK, _D), lambda i: (i, 0))],
        out_specs=pl.BlockSpec((2, _BK, _H), lambda i: (0, i, 0)),
        out_shape=jax.ShapeDtypeStruct((2, _N, _H), jnp.float32),
    )(h)


def kernel(h, edge_index, ppr_weight, W1, b1, W2, b2):
    pk3 = (edge_index[0] | (edge_index[1] << 16)).reshape(_T, _C, _K)
    ppr3 = ppr_weight.reshape(_T, _C, _K)

    aug = _prep(h)
    agg1 = _sc_agg(aug.reshape(2 * _N, _H), pk3, ppr3)
    hh1, aug1 = _tc_update(h, agg1, W1, b1, make_aux=True)
    agg2 = _sc_agg(aug1.reshape(2 * _N, _H), pk3, ppr3)
    (hh2,) = _tc_update(hh1, agg2, W2, b2, make_aux=False)
    return hh2


# speedup vs baseline: 5.8546x; 1.1543x over previous
import functools

import jax
import jax.numpy as jnp
from jax import lax
from jax.experimental import pallas as pl
from jax.experimental.pallas import tpu as pltpu
from jax.experimental.pallas import tpu_sc as plsc

_N = 10000
_E = 160000
_D = 256
_H = 144
_K = 80
_C = 125
_T = 16
_RPT = _N // _T
_BK = 1000


def _sc_agg(h_aug, pk3, ppr3):
    mesh = plsc.VectorSubcoreMesh(core_axis_name="c", subcore_axis_name="s")

    @functools.partial(
        pl.kernel,
        mesh=mesh,
        compiler_params=pltpu.CompilerParams(use_tc_tiling_on_sc=False),
        out_type=jax.ShapeDtypeStruct((2, _N, _H), jnp.float32),
        scratch_types=[
            pltpu.VMEM((3, _K), jnp.int32),
            pltpu.VMEM((3, _K), jnp.float32),
            pltpu.VMEM((3, _K), jnp.int32),
            pltpu.VMEM((3, _K), jnp.int32),
            pltpu.VMEM((3, _K, _H), jnp.float32),
            pltpu.VMEM_SHARED((_N, _H), jnp.float32),
            pltpu.SemaphoreType.DMA,
            pltpu.SemaphoreType.DMA,
            pltpu.SemaphoreType.DMA,
            pltpu.SemaphoreType.DMA,
            pltpu.SemaphoreType.DMA,
            pltpu.SemaphoreType.DMA,
            pltpu.SemaphoreType.DMA,
            pltpu.SemaphoreType.DMA,
            pltpu.SemaphoreType.DMA,
            pltpu.SemaphoreType.DMA,
            pltpu.SemaphoreType.DMA,
            pltpu.SemaphoreType.DMA,
        ],
    )
    def k(h_hbm, pk_hbm, ppr_hbm, out_hbm,
          pk_v, ppr_v, idx_v, dst_v, rows_v, acc_sh,
          ek0, ek1, ek2, ep0, ep1, ep2, gs0, gs1, gs2, ss0, ss1, ss2):
        c = lax.axis_index("c")
        s = lax.axis_index("s")
        cofs = c * _N
        eks = (ek0, ek1, ek2)
        eps = (ep0, ep1, ep2)
        gss = (gs0, gs1, gs2)
        sss = (ss0, ss1, ss2)

        def start_edge(t, e):
            pltpu.async_copy(pk_hbm.at[s, t], pk_v.at[e], eks[e])
            pltpu.async_copy(ppr_hbm.at[s, t], ppr_v.at[e], eps[e])

        def wait_edge(e):
            pltpu.make_async_copy(pk_hbm.at[s, 0], pk_v.at[e], eks[e]).wait()
            pltpu.make_async_copy(ppr_hbm.at[s, 0], ppr_v.at[e], eps[e]).wait()

        def unpack(e, b):
            for u in range(_K // 16):
                sl = pl.ds(u * 16, 16)
                pk = pk_v[e, sl]
                idx_v[b, sl] = (pk & 0xFFFF) + cofs
                dst_v[b, sl] = pk >> 16

        def start_gather(b):
            pltpu.async_copy(h_hbm.at[idx_v.at[b]], rows_v.at[b], gss[b])

        def wait_gather(b):
            pltpu.make_async_copy(
                h_hbm.at[idx_v.at[b]], rows_v.at[b], gss[b]).wait()

        def start_scatter(b):
            pltpu.async_copy(rows_v.at[b], acc_sh.at[dst_v.at[b]], sss[b],
                             add=True)

        def wait_scatter(b):
            pltpu.make_async_copy(
                rows_v.at[b], acc_sh.at[dst_v.at[b]], sss[b]).wait()

        def scale(e4, b):
            def sgroup(g, gcarry):
                pw = ppr_v[e4, pl.ds(g * 16, 16)]
                rbase = g * 16
                for rr in range(16):
                    pv = jnp.full((16,), pw[rr], jnp.float32)
                    for u in range(_H // 16):
                        sl = pl.ds(u * 16, 16)
                        rows_v[b, rbase + rr, sl] = (
                            rows_v[b, rbase + rr, sl] * pv)
                return gcarry

            lax.fori_loop(0, _K // 16, sgroup, 0)

        zero16f = jnp.zeros((16,), jnp.float32)
        zero16i = jnp.zeros((16,), jnp.int32)

        def zrow(r, carry):
            for b in (0, 2):
                for u in range(_H // 16):
                    rows_v[b, r, pl.ds(u * 16, 16)] = zero16f
            return carry

        lax.fori_loop(0, _K, zrow, 0)
        for u in range(_K // 16):
            dst_v[2, pl.ds(u * 16, 16)] = zero16i

        base = s * _RPT
        for blk in range(_RPT // _K):
            pltpu.sync_copy(rows_v.at[0], acc_sh.at[pl.ds(base + blk * _K, _K)])
        pltpu.sync_copy(rows_v.at[0, pl.ds(0, _RPT % _K)],
                        acc_sh.at[pl.ds(base + (_RPT // _K) * _K, _RPT % _K)])

        for t0 in range(3):
            start_edge(t0, t0)

        plsc.subcore_barrier()

        start_scatter(2)

        wait_edge(0)
        unpack(0, 0)
        start_gather(0)
        wait_edge(1)
        unpack(1, 1)
        start_gather(1)

        def step(t, m3, prep, estart):
            b = m3
            b2 = (m3 + 2) % 3
            if prep:
                wait_edge(b2)
                wait_scatter(b2)
                unpack(b2, b2)
                start_gather(b2)
            wait_gather(b)
            scale(b, b)
            if estart:

                @pl.when(t + 3 < _C)
                def _():
                    start_edge(t + 3, b)

            start_scatter(b)

        def triple(i, carry):
            t = i * 3
            step(t, 0, True, True)
            step(t + 1, 1, True, True)
            step(t + 2, 2, True, True)
            return carry

        lax.fori_loop(0, 40, triple, 0)

        step(120, 0, True, True)
        step(121, 1, True, True)
        step(122, 2, True, True)
        step(123, 0, False, False)
        step(124, 1, False, False)

        wait_scatter(2)
        wait_scatter(0)
        wait_scatter(1)

        plsc.subcore_barrier()

        _ZR = 125
        for blk in range(_RPT // _ZR):
            ofs = base + blk * _ZR
            pltpu.sync_copy(acc_sh.at[pl.ds(ofs, _ZR)],
                            out_hbm.at[c, pl.ds(ofs, _ZR)])

    return k(h_aug, pk3, ppr3)


def _aug_cols(bk):
    col = lax.broadcasted_iota(jnp.int32, (bk, _H - 128), 1)
    return jnp.where(col == 0, 1.0, 0.0).astype(jnp.float32)


def _tc_body(hh_ref, agg_ref, ws_ref, wl_ref, wh_ref, b_ref, o_ref,
             aux_ref=None):
    dn = (((1,), (1,)), ((), ()))
    w = agg_ref[0, :, 128:129]
    inv = 1.0 / jnp.where(w == 0.0, 1.0, w)
    alo = agg_ref[0, :, 0:128] * inv
    ahi = agg_ref[1, :, 0:128] * inv
    acc = (lax.dot_general(hh_ref[...], ws_ref[...], dn,
                           preferred_element_type=jnp.float32)
           + lax.dot_general(alo, wl_ref[...], dn,
                             preferred_element_type=jnp.float32)
           + lax.dot_general(ahi, wh_ref[...], dn,
                             preferred_element_type=jnp.float32)
           + b_ref[...])
    y = jnp.where(acc > 0.0, acc, 0.01 * acc)
    ss = jnp.sum(y * y, axis=1, keepdims=True)
    nrm = jnp.sqrt(ss)
    nrm = jnp.where(nrm == 0.0, 1.0, nrm)
    res = y / nrm
    o_ref[...] = res
    if aux_ref is not None:
        tail = _aug_cols(res.shape[0])
        aux_ref[0, :, 0:128] = res[:, 0:128]
        aux_ref[1, :, 0:128] = res[:, 128:256]
        aux_ref[0, :, 128:_H] = tail
        aux_ref[1, :, 128:_H] = tail


def _tc_update(hh, agg, W, b, make_aux):
    grid = (_N // _BK,)
    in_specs = [
        pl.BlockSpec((_BK, _D), lambda i: (i, 0)),
        pl.BlockSpec((2, _BK, _H), lambda i: (0, i, 0)),
        pl.BlockSpec((_D, _D), lambda i: (0, 0)),
        pl.BlockSpec((_D, 128), lambda i: (0, 2)),
        pl.BlockSpec((_D, 128), lambda i: (0, 3)),
        pl.BlockSpec((1, _D), lambda i: (0, 0)),
    ]
    out_shape = [jax.ShapeDtypeStruct((_N, _D), jnp.float32)]
    out_specs = [pl.BlockSpec((_BK, _D), lambda i: (i, 0))]
    if make_aux:
        out_shape.append(jax.ShapeDtypeStruct((2, _N, _H), jnp.float32))
        out_specs.append(pl.BlockSpec((2, _BK, _H), lambda i: (0, i, 0)))
    return pl.pallas_call(
        _tc_body,
        grid=grid,
        in_specs=in_specs,
        out_specs=out_specs,
        out_shape=out_shape,
    )(hh, agg, W, W, W, b.reshape(1, _D))


def _prep_body(h_ref, o_ref):
    tail = _aug_cols(h_ref.shape[0])
    o_ref[0, :, 0:128] = h_ref[:, 0:128]
    o_ref[1, :, 0:128] = h_ref[:, 128:256]
    o_ref[0, :, 128:_H] = tail
    o_ref[1, :, 128:_H] = tail


def _prep(h):
    return pl.pallas_call(
        _prep_body,
        grid=(_N // _BK,),
        in_specs=[pl.BlockSpec((_BK, _D), lambda i: (i, 0))],
        out_specs=pl.BlockSpec((2, _BK, _H), lambda i: (0, i, 0)),
        out_shape=jax.ShapeDtypeStruct((2, _N, _H), jnp.float32),
    )(h)


def kernel(h, edge_index, ppr_weight, W1, b1, W2, b2):
    pk3 = (edge_index[0] | (edge_index[1] << 16)).reshape(_T, _C, _K)
    ppr3 = ppr_weight.reshape(_T, _C, _K)

    aug = _prep(h)
    agg1 = _sc_agg(aug.reshape(2 * _N, _H), pk3, ppr3)
    hh1, aug1 = _tc_update(h, agg1, W1, b1, make_aux=True)
    agg2 = _sc_agg(aug1.reshape(2 * _N, _H), pk3, ppr3)
    (hh2,) = _tc_update(hh1, agg2, W2, b2, make_aux=False)
    return hh2
